# baseline (speedup 1.0000x reference)
import functools

import jax
import jax.numpy as jnp
from jax import lax
from jax.experimental import pallas as pl
from jax.experimental.pallas import tpu as pltpu
from jax.experimental.pallas import tpu_sc as plsc

N = 10000
E = 160000
D = 256
Q = 64
NQ = 4
NR = 2
NC = 2
NS = 16
NW = NC * NS

HIST = 10240
ET = E // NS
CH = 80
K = 125
NP = 10240
RT = NP // NS
WB = 80
NB = 4


def _mesh():
    return plsc.VectorSubcoreMesh(
        core_axis_name="c", subcore_axis_name="s", num_cores=NC, num_subcores=NS
    )


def _deg_body(col_hbm, out_hbm, colv, hist, acc, tmp, hists_sh):
    c = lax.axis_index("c")
    s = lax.axis_index("s")
    pltpu.sync_copy(col_hbm.at[s], colv)
    z16 = jnp.zeros((16,), jnp.float32)
    one16 = jnp.ones((16,), jnp.float32)

    def zero(i, carry):
        hist[pl.ds(i * 16, 16)] = z16
        return carry

    lax.fori_loop(0, HIST // 16, zero, 0)

    def upd(i, carry):
        idx = colv[pl.ds(i * 16, 16)]
        plsc.addupdate_scatter(hist, [idx], one16)
        return carry

    lax.fori_loop(0, ET // 16, upd, 0)
    pltpu.sync_copy(hist, hists_sh.at[pl.ds(s * HIST, HIST)])
    plsc.subcore_barrier()

    @pl.when(c == 0)
    def _():
        def zacc(i, carry):
            acc[pl.ds(i * 16, 16)] = z16
            return carry

        lax.fori_loop(0, RT // 16, zacc, 0)

        def red(t2, carry):
            pltpu.sync_copy(hists_sh.at[pl.ds(t2 * HIST + s * RT, RT)], tmp)

            def addv(i, carry2):
                acc[pl.ds(i * 16, 16)] += tmp[pl.ds(i * 16, 16)]
                return carry2

            return lax.fori_loop(0, RT // 16, addv, carry)

        lax.fori_loop(0, NS, red, 0)
        pltpu.sync_copy(acc, out_hbm.at[pl.ds(s * RT, RT)])


def _deg(col_t):
    fn = pl.kernel(
        _deg_body,
        out_type=jax.ShapeDtypeStruct((HIST,), jnp.float32),
        mesh=_mesh(),
        scratch_types=[
            pltpu.VMEM((ET,), jnp.int32),
            pltpu.VMEM((HIST,), jnp.float32),
            pltpu.VMEM((RT,), jnp.float32),
            pltpu.VMEM((RT,), jnp.float32),
            pltpu.VMEM_SHARED((NS * HIST,), jnp.float32),
        ],
        compiler_params=pltpu.CompilerParams(needs_layout_passes=False),
    )
    return fn(col_t)


_RB = 1000
_GRID = N // _RB


def _mm_body(x_ref, w_ref, deg_ref, t_ref, wt_ref, bt_ref,
             y_ref, dinv_ref, temb_ref):
    dinv = lax.rsqrt(deg_ref[...] + 1.0)
    dinv_ref[...] = dinv
    xw = jnp.dot(x_ref[...], w_ref[...], preferred_element_type=jnp.float32)
    y = xw * dinv
    y_ref[0] = y[:, 0 * Q:1 * Q]
    y_ref[1] = y[:, 1 * Q:2 * Q]
    y_ref[2] = y[:, 2 * Q:3 * Q]
    y_ref[3] = y[:, 3 * Q:4 * Q]

    @pl.when(pl.program_id(0) == 0)
    def _():
        te = jnp.dot(t_ref[...], wt_ref[...], preferred_element_type=jnp.float32)
        temb_ref[...] = jnp.maximum(te + bt_ref[...], 0.0)


def _mm(x, W_gcn, deg_col, t, W_t, bt2):
    return pl.pallas_call(
        _mm_body,
        grid=(_GRID,),
        in_specs=[
            pl.BlockSpec((_RB, D), lambda j: (j, 0)),
            pl.BlockSpec((D, D), lambda j: (0, 0)),
            pl.BlockSpec((_RB, 1), lambda j: (j, 0)),
            pl.BlockSpec((1, D), lambda j: (0, 0)),
            pl.BlockSpec((D, D), lambda j: (0, 0)),
            pl.BlockSpec((1, D), lambda j: (0, 0)),
        ],
        out_specs=[
            pl.BlockSpec((NQ, _RB, Q), lambda j: (0, j, 0)),
            pl.BlockSpec((_RB, 1), lambda j: (j, 0)),
            pl.BlockSpec((1, D), lambda j: (0, 0)),
        ],
        out_shape=[
            jax.ShapeDtypeStruct((NQ, N, Q), jnp.float32),
            jax.ShapeDtypeStruct((N, 1), jnp.float32),
            jax.ShapeDtypeStruct((1, D), jnp.float32),
        ],
    )(x, W_gcn, deg_col, t, W_t, bt2)


def _scat_body(y_hbm, row_hbm, col_hbm, out_hbm,
               rowi, coli, buf, zbuf, wbuf, acc_sh, *sems):
    semg = sems[:NB]
    semsc = sems[NB:]
    c = lax.axis_index("c")
    s = lax.axis_index("s")
    pltpu.sync_copy(row_hbm.at[s], rowi)
    pltpu.sync_copy(col_hbm.at[s], coli)

    z16 = jnp.zeros((16,), jnp.float32)

    def zrow(i, carry):
        def zcol(k, carry2):
            zbuf[i, pl.ds(k * 16, 16)] = z16
            return carry2
        return lax.fori_loop(0, Q // 16, zcol, carry)

    lax.fori_loop(0, WB, zrow, 0)

    for p in range(NR):
        q = NR * c + p
        yt = y_hbm.at[q]

        def zacc(k, carry):
            pltpu.sync_copy(zbuf, acc_sh.at[pl.ds(s * RT + k * WB, WB)])
            return carry

        lax.fori_loop(0, RT // WB, zacc, 0)
        plsc.subcore_barrier()

        pltpu.async_copy(yt.at[rowi.at[0]], buf.at[0], semg[0])

        def lap(i, carry):
            j0 = 2 * i
            j1 = 2 * i + 1
            pltpu.async_copy(yt.at[rowi.at[j1]], buf.at[1], semg[1])
            pltpu.make_async_copy(yt.at[rowi.at[j0]], buf.at[0], semg[0]).wait()
            pltpu.sync_copy(buf.at[0], acc_sh.at[coli.at[j0]], add=True)

            @pl.when(i < CH // 2 - 1)
            def _():
                pltpu.async_copy(yt.at[rowi.at[j0 + 2]], buf.at[0], semg[0])

            pltpu.make_async_copy(yt.at[rowi.at[j1]], buf.at[1], semg[1]).wait()
            pltpu.sync_copy(buf.at[1], acc_sh.at[coli.at[j1]], add=True)
            return carry

        lax.fori_loop(0, CH // 2, lap, 0)
        plsc.subcore_barrier()

        def wb(k, carry):
            off = s * RT + k * WB

            @pl.when(off < N)
            def _():
                pltpu.sync_copy(acc_sh.at[pl.ds(off, WB)], wbuf)
                pltpu.sync_copy(wbuf, out_hbm.at[q].at[pl.ds(off, WB)])

            return carry

        lax.fori_loop(0, RT // WB, wb, 0)


def _scatter(y, row_b, col_b):
    fn = pl.kernel(
        _scat_body,
        out_type=jax.ShapeDtypeStruct((NQ, N, Q), jnp.float32),
        mesh=_mesh(),
        scratch_types=[
            pltpu.VMEM((CH, K), jnp.int32),
            pltpu.VMEM((CH, K), jnp.int32),
            pltpu.VMEM((NB, K, Q), jnp.float32),
            pltpu.VMEM((WB, Q), jnp.float32),
            pltpu.VMEM((WB, Q), jnp.float32),
            pltpu.VMEM_SHARED((NP, Q), jnp.float32),
        ] + [pltpu.SemaphoreType.DMA] * (2 * NB),
        compiler_params=pltpu.CompilerParams(
            needs_layout_passes=False, use_tc_tiling_on_sc=False
        ),
    )
    return fn(y, row_b, col_b)


def _fin_body(acc_ref, y_ref, dinv_ref, temb_ref, bg_ref, g_ref, b_ref,
              o_ref, s1s, s2s):
    p = pl.program_id(0)
    j = pl.program_id(1)
    a = jnp.concatenate([acc_ref[0], acc_ref[1], acc_ref[2], acc_ref[3]], axis=1)
    yy = jnp.concatenate([y_ref[0], y_ref[1], y_ref[2], y_ref[3]], axis=1)
    h = dinv_ref[...] * (a + yy) + bg_ref[...] + temb_ref[...]
    hr = jnp.maximum(h, 0.0)

    @pl.when((p == 0) & (j == 0))
    def _():
        s1s[...] = jnp.zeros_like(s1s)
        s2s[...] = jnp.zeros_like(s2s)

    @pl.when(p == 0)
    def _():
        s1s[...] += jnp.sum(hr, axis=0, keepdims=True)
        s2s[...] += jnp.sum(hr * hr, axis=0, keepdims=True)

    @pl.when(p == 1)
    def _():
        mean = s1s[...] * (1.0 / N)
        var = s2s[...] * (1.0 / N) - mean * mean
        sc = g_ref[...] * lax.rsqrt(var + 1e-5)
        o_ref[...] = (hr - mean) * sc + b_ref[...]


def _fin(acc, y, dinv_col, temb, bg2, g2, be2):
    return pl.pallas_call(
        _fin_body,
        grid=(2, _GRID),
        in_specs=[
            pl.BlockSpec((NQ, _RB, Q), lambda p, j: (0, j, 0)),
            pl.BlockSpec((NQ, _RB, Q), lambda p, j: (0, j, 0)),
            pl.BlockSpec((_RB, 1), lambda p, j: (j, 0)),
            pl.BlockSpec((1, D), lambda p, j: (0, 0)),
            pl.BlockSpec((1, D), lambda p, j: (0, 0)),
            pl.BlockSpec((1, D), lambda p, j: (0, 0)),
            pl.BlockSpec((1, D), lambda p, j: (0, 0)),
        ],
        out_specs=pl.BlockSpec((_RB, D), lambda p, j: (p * j, 0)),
        out_shape=jax.ShapeDtypeStruct((N, D), jnp.float32),
        scratch_shapes=[
            pltpu.VMEM((1, D), jnp.float32),
            pltpu.VMEM((1, D), jnp.float32),
        ],
    )(acc, y, dinv_col, temb, bg2, g2, be2)


def kernel(x, edge_index, t, W_gcn, b_gcn, W_t, b_t, gamma, beta):
    row = edge_index[0]
    col = edge_index[1]

    col_t = col.reshape(NS, ET)
    row_b = row.reshape(NS, CH, K)
    col_b = col.reshape(NS, CH, K)

    degs = _deg(col_t)
    deg_col = degs[:N].reshape(N, 1)

    bt2 = b_t.reshape(1, D)
    bg2 = b_gcn.reshape(1, D)
    g2 = gamma.reshape(1, D)
    be2 = beta.reshape(1, D)

    y, dinv_col, temb = _mm(x, W_gcn, deg_col, t, W_t, bt2)
    acc = _scatter(y, row_b, col_b)
    return _fin(acc, y, dinv_col, temb, bg2, g2, be2)


# speedup vs baseline: 1.0187x; 1.0187x over previous
import functools

import jax
import jax.numpy as jnp
from jax import lax
from jax.experimental import pallas as pl
from jax.experimental.pallas import tpu as pltpu
from jax.experimental.pallas import tpu_sc as plsc

N = 10000
E = 160000
D = 256
Q = 64
NQ = 4
NR = 2
NC = 2
NS = 16
NW = NC * NS

HIST = 10240
ET = E // NS
CH = 80
K = 125
NP = 10240
RT = NP // NS
WB = 80
NB = 4


def _mesh():
    return plsc.VectorSubcoreMesh(
        core_axis_name="c", subcore_axis_name="s", num_cores=NC, num_subcores=NS
    )


def _deg_body(col_hbm, out_hbm, colv, hist, acc, tmp, hists_sh):
    c = lax.axis_index("c")
    s = lax.axis_index("s")
    pltpu.sync_copy(col_hbm.at[s], colv)
    z16 = jnp.zeros((16,), jnp.float32)
    one16 = jnp.ones((16,), jnp.float32)

    def zero(i, carry):
        hist[pl.ds(i * 16, 16)] = z16
        return carry

    lax.fori_loop(0, HIST // 16, zero, 0)

    def upd(i, carry):
        idx = colv[pl.ds(i * 16, 16)]
        plsc.addupdate_scatter(hist, [idx], one16)
        return carry

    lax.fori_loop(0, ET // 16, upd, 0)
    pltpu.sync_copy(hist, hists_sh.at[pl.ds(s * HIST, HIST)])
    plsc.subcore_barrier()

    @pl.when(c == 0)
    def _():
        def zacc(i, carry):
            acc[pl.ds(i * 16, 16)] = z16
            return carry

        lax.fori_loop(0, RT // 16, zacc, 0)

        def red(t2, carry):
            pltpu.sync_copy(hists_sh.at[pl.ds(t2 * HIST + s * RT, RT)], tmp)

            def addv(i, carry2):
                acc[pl.ds(i * 16, 16)] += tmp[pl.ds(i * 16, 16)]
                return carry2

            return lax.fori_loop(0, RT // 16, addv, carry)

        lax.fori_loop(0, NS, red, 0)
        pltpu.sync_copy(acc, out_hbm.at[pl.ds(s * RT, RT)])


def _deg(col_t):
    fn = pl.kernel(
        _deg_body,
        out_type=jax.ShapeDtypeStruct((HIST,), jnp.float32),
        mesh=_mesh(),
        scratch_types=[
            pltpu.VMEM((ET,), jnp.int32),
            pltpu.VMEM((HIST,), jnp.float32),
            pltpu.VMEM((RT,), jnp.float32),
            pltpu.VMEM((RT,), jnp.float32),
            pltpu.VMEM_SHARED((NS * HIST,), jnp.float32),
        ],
        compiler_params=pltpu.CompilerParams(needs_layout_passes=False),
    )
    return fn(col_t)


_RB = 1000
_GRID = N // _RB


def _mm_body(x_ref, w_ref, deg_ref, t_ref, wt_ref, bt_ref,
             y_ref, dinv_ref, temb_ref):
    dinv = lax.rsqrt(deg_ref[...] + 1.0)
    dinv_ref[...] = dinv
    xw = jnp.dot(x_ref[...], w_ref[...], preferred_element_type=jnp.float32)
    y = xw * dinv
    y_ref[0] = y[:, 0 * Q:1 * Q]
    y_ref[1] = y[:, 1 * Q:2 * Q]
    y_ref[2] = y[:, 2 * Q:3 * Q]
    y_ref[3] = y[:, 3 * Q:4 * Q]

    @pl.when(pl.program_id(0) == 0)
    def _():
        te = jnp.dot(t_ref[...], wt_ref[...], preferred_element_type=jnp.float32)
        temb_ref[...] = jnp.maximum(te + bt_ref[...], 0.0)


def _mm(x, W_gcn, deg_col, t, W_t, bt2):
    return pl.pallas_call(
        _mm_body,
        grid=(_GRID,),
        in_specs=[
            pl.BlockSpec((_RB, D), lambda j: (j, 0)),
            pl.BlockSpec((D, D), lambda j: (0, 0)),
            pl.BlockSpec((_RB, 1), lambda j: (j, 0)),
            pl.BlockSpec((1, D), lambda j: (0, 0)),
            pl.BlockSpec((D, D), lambda j: (0, 0)),
            pl.BlockSpec((1, D), lambda j: (0, 0)),
        ],
        out_specs=[
            pl.BlockSpec((NQ, _RB, Q), lambda j: (0, j, 0)),
            pl.BlockSpec((_RB, 1), lambda j: (j, 0)),
            pl.BlockSpec((1, D), lambda j: (0, 0)),
        ],
        out_shape=[
            jax.ShapeDtypeStruct((NQ, N, Q), jnp.float32),
            jax.ShapeDtypeStruct((N, 1), jnp.float32),
            jax.ShapeDtypeStruct((1, D), jnp.float32),
        ],
    )(x, W_gcn, deg_col, t, W_t, bt2)


def _scat_body(y_hbm, row_hbm, col_hbm, out_hbm,
               rowi, coli, buf, zbuf, wbuf, acc_sh, *sems):
    semg = sems[:NB]
    semsc = sems[NB:]
    c = lax.axis_index("c")
    s = lax.axis_index("s")
    pltpu.sync_copy(row_hbm.at[s], rowi)
    pltpu.sync_copy(col_hbm.at[s], coli)

    z16 = jnp.zeros((16,), jnp.float32)

    def zrow(i, carry):
        def zcol(k, carry2):
            zbuf[i, pl.ds(k * 16, 16)] = z16
            return carry2
        return lax.fori_loop(0, Q // 16, zcol, carry)

    lax.fori_loop(0, WB, zrow, 0)

    for p in range(NR):
        q = NR * c + p
        yt = y_hbm.at[q]

        def zacc(k, carry):
            off = s * RT + k * WB

            @pl.when(off < N)
            def _():
                pltpu.sync_copy(yt.at[pl.ds(off, WB)], wbuf)
                pltpu.sync_copy(wbuf, acc_sh.at[pl.ds(off, WB)])

            @pl.when(off >= N)
            def _():
                pltpu.sync_copy(zbuf, acc_sh.at[pl.ds(off, WB)])

            return carry

        lax.fori_loop(0, RT // WB, zacc, 0)
        plsc.subcore_barrier()

        pltpu.async_copy(yt.at[rowi.at[0]], buf.at[0], semg[0])
        pltpu.async_copy(yt.at[rowi.at[1]], buf.at[1], semg[1])

        def lap(i, carry):
            for b in range(NB):
                j = NB * i + b
                b2 = (b + 2) % NB
                pltpu.make_async_copy(
                    yt.at[rowi.at[j]], buf.at[b], semg[b]
                ).wait()
                pltpu.async_copy(
                    buf.at[b], acc_sh.at[coli.at[j]], semsc[b], add=True
                )

                @pl.when(j >= 2)
                def _():
                    pltpu.make_async_copy(
                        buf.at[b2], acc_sh.at[coli.at[j - 2]], semsc[b2]
                    ).wait()

                @pl.when(j + 2 < CH)
                def _():
                    pltpu.async_copy(
                        yt.at[rowi.at[j + 2]], buf.at[b2], semg[b2]
                    )

            return carry

        lax.fori_loop(0, CH // NB, lap, 0)
        pltpu.make_async_copy(
            buf.at[(CH - 2) % NB], acc_sh.at[coli.at[CH - 2]], semsc[(CH - 2) % NB]
        ).wait()
        pltpu.make_async_copy(
            buf.at[(CH - 1) % NB], acc_sh.at[coli.at[CH - 1]], semsc[(CH - 1) % NB]
        ).wait()
        plsc.subcore_barrier()

        def wb(k, carry):
            off = s * RT + k * WB

            @pl.when(off < N)
            def _():
                pltpu.sync_copy(acc_sh.at[pl.ds(off, WB)], wbuf)
                pltpu.sync_copy(wbuf, out_hbm.at[q].at[pl.ds(off, WB)])

            return carry

        lax.fori_loop(0, RT // WB, wb, 0)


def _scatter(y, row_b, col_b):
    fn = pl.kernel(
        _scat_body,
        out_type=jax.ShapeDtypeStruct((NQ, N, Q), jnp.float32),
        mesh=_mesh(),
        scratch_types=[
            pltpu.VMEM((CH, K), jnp.int32),
            pltpu.VMEM((CH, K), jnp.int32),
            pltpu.VMEM((NB, K, Q), jnp.float32),
            pltpu.VMEM((WB, Q), jnp.float32),
            pltpu.VMEM((WB, Q), jnp.float32),
            pltpu.VMEM_SHARED((NP, Q), jnp.float32),
        ] + [pltpu.SemaphoreType.DMA] * (2 * NB),
        compiler_params=pltpu.CompilerParams(
            needs_layout_passes=False, use_tc_tiling_on_sc=False
        ),
    )
    return fn(y, row_b, col_b)


def _fin_body(acc_ref, dinv_ref, temb_ref, bg_ref, g_ref, b_ref,
              o_ref, s1s, s2s):
    p = pl.program_id(0)
    j = pl.program_id(1)
    a = jnp.concatenate([acc_ref[0], acc_ref[1], acc_ref[2], acc_ref[3]], axis=1)
    h = dinv_ref[...] * a + bg_ref[...] + temb_ref[...]
    hr = jnp.maximum(h, 0.0)

    @pl.when((p == 0) & (j == 0))
    def _():
        s1s[...] = jnp.zeros_like(s1s)
        s2s[...] = jnp.zeros_like(s2s)

    @pl.when(p == 0)
    def _():
        s1s[...] += jnp.sum(hr, axis=0, keepdims=True)
        s2s[...] += jnp.sum(hr * hr, axis=0, keepdims=True)

    @pl.when(p == 1)
    def _():
        mean = s1s[...] * (1.0 / N)
        var = s2s[...] * (1.0 / N) - mean * mean
        sc = g_ref[...] * lax.rsqrt(var + 1e-5)
        o_ref[...] = (hr - mean) * sc + b_ref[...]


def _fin(acc, dinv_col, temb, bg2, g2, be2):
    return pl.pallas_call(
        _fin_body,
        grid=(2, _GRID),
        in_specs=[
            pl.BlockSpec((NQ, _RB, Q), lambda p, j: (0, j, 0)),
            pl.BlockSpec((_RB, 1), lambda p, j: (j, 0)),
            pl.BlockSpec((1, D), lambda p, j: (0, 0)),
            pl.BlockSpec((1, D), lambda p, j: (0, 0)),
            pl.BlockSpec((1, D), lambda p, j: (0, 0)),
            pl.BlockSpec((1, D), lambda p, j: (0, 0)),
        ],
        out_specs=pl.BlockSpec((_RB, D), lambda p, j: (p * j, 0)),
        out_shape=jax.ShapeDtypeStruct((N, D), jnp.float32),
        scratch_shapes=[
            pltpu.VMEM((1, D), jnp.float32),
            pltpu.VMEM((1, D), jnp.float32),
        ],
    )(acc, dinv_col, temb, bg2, g2, be2)


def kernel(x, edge_index, t, W_gcn, b_gcn, W_t, b_t, gamma, beta):
    row = edge_index[0]
    col = edge_index[1]

    col_t = col.reshape(NS, ET)
    row_b = row.reshape(NS, CH, K)
    col_b = col.reshape(NS, CH, K)

    degs = _deg(col_t)
    deg_col = degs[:N].reshape(N, 1)

    bt2 = b_t.reshape(1, D)
    bg2 = b_gcn.reshape(1, D)
    g2 = gamma.reshape(1, D)
    be2 = beta.reshape(1, D)

    y, dinv_col, temb = _mm(x, W_gcn, deg_col, t, W_t, bt2)
    acc = _scatter(y, row_b, col_b)
    return _fin(acc, dinv_col, temb, bg2, g2, be2)
